# Initial kernel scaffold; baseline (speedup 1.0000x reference)
import jax
import jax.numpy as jnp
from jax.experimental import pallas as pl


def kernel(inputs, table):
    raise NotImplementedError("write your pallas kernel here")


# speedup vs baseline: 4.0884x; 4.0884x over previous
import functools

import jax
import jax.numpy as jnp
from jax import lax
from jax.experimental import pallas as pl
from jax.experimental.pallas import tpu as pltpu
from jax.experimental.pallas import tpu_sc as plsc

VOCAB = 100000
EMBED_DIM = 64
BATCH = 4096
HIST = 50

NUM_CORES = 2
NUM_SUBCORES = 16
NUM_WORKERS = NUM_CORES * NUM_SUBCORES

CHUNK = 128
TOTAL = BATCH * HIST
CHUNKS_PER_W = TOTAL // (NUM_WORKERS * CHUNK)


PER_W = TOTAL // NUM_WORKERS


def _sc_gather(table, idx_flat):
    mesh = plsc.VectorSubcoreMesh(core_axis_name="c", subcore_axis_name="s")

    @functools.partial(
        pl.kernel,
        mesh=mesh,
        out_type=jax.ShapeDtypeStruct((TOTAL, EMBED_DIM), jnp.float32),
        scratch_types=[
            pltpu.VMEM((PER_W,), jnp.int32),
            pltpu.VMEM((CHUNK, EMBED_DIM), jnp.float32),
            pltpu.SemaphoreType.DMA,
        ],
        compiler_params=pltpu.CompilerParams(use_tc_tiling_on_sc=False),
    )
    def k(table_hbm, idx_hbm, out_hbm, idx_v, rows_v, gsem):
        wid = lax.axis_index("s") * NUM_CORES + lax.axis_index("c")
        base = wid * PER_W
        pltpu.sync_copy(idx_hbm.at[pl.ds(base, PER_W)], idx_v)

        def step(g, carry):
            off = pl.multiple_of(g * CHUNK, CHUNK)
            pltpu.async_copy(
                table_hbm.at[idx_v.at[pl.ds(off, CHUNK)]], rows_v, gsem
            ).wait()
            pltpu.sync_copy(rows_v, out_hbm.at[pl.ds(base + off, CHUNK)])
            return carry

        lax.fori_loop(0, CHUNKS_PER_W, step, 0)

    return k(table, idx_flat)


def kernel(inputs, table):
    idx_flat = inputs.astype(jnp.int32).reshape(TOTAL)
    flat = _sc_gather(table, idx_flat)
    return flat.reshape(BATCH, HIST, EMBED_DIM)


# speedup vs baseline: 4.5927x; 1.1233x over previous
import functools

import jax
import jax.numpy as jnp
from jax import lax
from jax.experimental import pallas as pl
from jax.experimental.pallas import tpu as pltpu
from jax.experimental.pallas import tpu_sc as plsc

VOCAB = 100000
EMBED_DIM = 64
BATCH = 4096
HIST = 50

NUM_CORES = 2
NUM_SUBCORES = 16
NUM_WORKERS = NUM_CORES * NUM_SUBCORES

CHUNK = 128
TOTAL = BATCH * HIST
CHUNKS_PER_W = TOTAL // (NUM_WORKERS * CHUNK)


PER_W = TOTAL // NUM_WORKERS
CHUNKS_PER_FILL = 5
FILL = CHUNK * CHUNKS_PER_FILL
FILLS = PER_W // FILL


def _sc_gather(table, idx_flat):
    mesh = plsc.VectorSubcoreMesh(core_axis_name="c", subcore_axis_name="s")

    @functools.partial(
        pl.kernel,
        mesh=mesh,
        out_type=jax.ShapeDtypeStruct((TOTAL, EMBED_DIM), jnp.float32),
        scratch_types=[
            pltpu.VMEM((PER_W,), jnp.int32),
            pltpu.VMEM((FILL, EMBED_DIM), jnp.float32),
            pltpu.VMEM((FILL, EMBED_DIM), jnp.float32),
            pltpu.SemaphoreType.DMA,
            pltpu.SemaphoreType.DMA,
            pltpu.SemaphoreType.DMA,
            pltpu.SemaphoreType.DMA,
        ],
        compiler_params=pltpu.CompilerParams(use_tc_tiling_on_sc=False),
    )
    def k(table_hbm, idx_hbm, out_hbm, idx_v, buf0, buf1, g0, g1, o0, o1):
        wid = lax.axis_index("s") * NUM_CORES + lax.axis_index("c")
        base = wid * PER_W
        bufs, gsems, osems = (buf0, buf1), (g0, g1), (o0, o1)
        pltpu.sync_copy(idx_hbm.at[pl.ds(base, PER_W)], idx_v)

        def fire_gather(f, b):
            for j in range(CHUNKS_PER_FILL):
                pltpu.async_copy(
                    table_hbm.at[idx_v.at[pl.ds(f * FILL + j * CHUNK, CHUNK)]],
                    bufs[b].at[pl.ds(j * CHUNK, CHUNK)],
                    gsems[b],
                )

        def drain_gather(b):
            pltpu.make_async_copy(
                table_hbm.at[pl.ds(0, FILL)], bufs[b], gsems[b]
            ).wait()

        def fire_out(f, b):
            pltpu.async_copy(
                bufs[b], out_hbm.at[pl.ds(base + f * FILL, FILL)], osems[b]
            )

        def drain_out(b):
            pltpu.make_async_copy(
                bufs[b], out_hbm.at[pl.ds(0, FILL)], osems[b]
            ).wait()

        fire_gather(0, 0)

        def body(g, carry):
            for b in range(2):
                f = g + b
                drain_gather(b)
                fire_out(f, b)
                nb = 1 - b

                @pl.when(f >= 1)
                def _():
                    drain_out(nb)

                @pl.when(f + 1 < FILLS)
                def _():
                    fire_gather(f + 1, nb)

            return carry

        lax.fori_loop(0, FILLS // 2, lambda i, c: body(i * 2, c), 0)
        drain_out((FILLS - 1) % 2)

    return k(table, idx_flat)


def kernel(inputs, table):
    idx_flat = inputs.astype(jnp.int32).reshape(TOTAL)
    flat = _sc_gather(table, idx_flat)
    return flat.reshape(BATCH, HIST, EMBED_DIM)
